# baseline (speedup 1.0000x reference)
import functools

import jax
import jax.numpy as jnp
from jax import lax
from jax.experimental import pallas as pl
from jax.experimental.pallas import tpu as pltpu
from jax.experimental.pallas import tpu_sc as plsc

N_NODE = 10000
HID = 128
E_TOTAL = 320000
NUM_CORES = 2
NUM_SUBCORES = 16
NUM_WORKERS = NUM_CORES * NUM_SUBCORES
ROWS_PER_TILE = 632
N_ACC = NUM_SUBCORES * ROWS_PER_TILE
CHUNK = 128
EDGES_PER_WORKER = 10112
NUM_CHUNKS = EDGES_PER_WORKER // CHUNK
E_PAD = NUM_WORKERS * EDGES_PER_WORKER - E_TOTAL


def _enc_body(x_ref, w1_ref, b1_ref, w2_ref, b2_ref, w3_ref, b3_ref, out_ref,
              *, n_pts):
    x = x_ref[...]
    c = x[:, n_pts - 1:n_pts, :]
    col = lax.broadcasted_iota(jnp.int32, (1, 1, x.shape[-1]), 2)
    x = x - jnp.where(col < 2, c, 0.0)
    w1 = w1_ref[...]
    b1 = b1_ref[...]
    w2 = w2_ref[...]
    b2 = b2_ref[...]
    m = None
    for p in range(n_pts):
        h = jnp.dot(x[:, p, :], w1, preferred_element_type=jnp.float32) + b1
        h = jnp.maximum(h, 0.0)
        h = jnp.dot(h, w2, preferred_element_type=jnp.float32) + b2
        h = jnp.maximum(h, 0.0)
        m = h if m is None else jnp.maximum(m, h)
    z = jnp.dot(m, w3_ref[...], preferred_element_type=jnp.float32) + b3_ref[...]
    out_ref[...] = jnp.maximum(z, 0.0)


def _encode(x, w1, b1, w2, b2, w3, b3, block):
    n, n_pts, f_in = x.shape
    return pl.pallas_call(
        functools.partial(_enc_body, n_pts=n_pts),
        grid=(n // block,),
        in_specs=[
            pl.BlockSpec((block, n_pts, f_in), lambda i: (i, 0, 0)),
            pl.BlockSpec((f_in, HID), lambda i: (0, 0)),
            pl.BlockSpec((HID,), lambda i: (0,)),
            pl.BlockSpec((HID, HID), lambda i: (0, 0)),
            pl.BlockSpec((HID,), lambda i: (0,)),
            pl.BlockSpec((HID, HID), lambda i: (0, 0)),
            pl.BlockSpec((HID,), lambda i: (0,)),
        ],
        out_specs=pl.BlockSpec((block, HID), lambda i: (i, 0)),
        out_shape=jax.ShapeDtypeStruct((n, HID), jnp.float32),
    )(x, w1, b1, w2, b2, w3, b3)


def _agg_body(table_hbm, esrc_hbm, edst_hbm, zeros_hbm, ones_hbm, out_hbm,
              src0_v, src1_v, dst0_v, dst1_v, dst2_v, dst3_v, rows0_v,
              rows1_v, acc_sh, si0, si1, sg0, sg1, ss0, ss1, ss2, ss3):
    cid = lax.axis_index("c")
    sid = lax.axis_index("s")
    wid = sid * NUM_CORES + cid
    row0 = sid * ROWS_PER_TILE
    ebase = wid * EDGES_PER_WORKER

    src_v = (src0_v, src1_v)
    dst_v = (dst0_v, dst1_v)
    rows_v = (rows0_v, rows1_v)
    si = (si0, si1)
    sg = (sg0, sg1)

    def fire_idx(g, b, with_src):
        off = ebase + g * CHUNK
        if with_src:
            pltpu.async_copy(esrc_hbm.at[pl.ds(off, CHUNK)], src_v[b], si[b])
        pltpu.async_copy(edst_hbm.at[pl.ds(off, CHUNK)], dst_v[b], si[b])

    def wait_idx(b, with_src):
        if with_src:
            pltpu.make_async_copy(esrc_hbm.at[pl.ds(0, CHUNK)], src_v[b],
                                  si[b]).wait()
        pltpu.make_async_copy(edst_hbm.at[pl.ds(0, CHUNK)], dst_v[b],
                              si[b]).wait()

    def fire_gather(b):
        pltpu.async_copy(table_hbm.at[src_v[b]], rows_v[b], sg[b])

    def wait_gather(b):
        pltpu.make_async_copy(table_hbm.at[src_v[b]], rows_v[b], sg[b]).wait()

    pltpu.sync_copy(zeros_hbm.at[pl.ds(row0, ROWS_PER_TILE)],
                    acc_sh.at[pl.ds(row0, ROWS_PER_TILE)])
    plsc.subcore_barrier()

    fire_idx(0, 0, True)
    wait_idx(0, True)
    fire_gather(0)
    fire_idx(1, 1, True)

    def step1(g, b):
        nb = 1 - b
        wait_idx(nb, True)
        fire_gather(nb)
        wait_gather(b)
        pltpu.sync_copy(rows_v[b], acc_sh.at[dst_v[b]], add=True)

        @pl.when(g + 2 < NUM_CHUNKS)
        def _():
            fire_idx(g + 2, b, True)

    def outer1(o, carry):
        step1(2 * o, 0)
        step1(2 * o + 1, 1)
        return carry

    lax.fori_loop(0, (NUM_CHUNKS - 1) // 2, outer1, 0)
    last = (NUM_CHUNKS - 1) % 2
    wait_gather(last)
    pltpu.sync_copy(rows_v[last], acc_sh.at[dst_v[last]], add=True)

    plsc.subcore_barrier()
    pltpu.sync_copy(acc_sh.at[pl.ds(row0, ROWS_PER_TILE)],
                    out_hbm.at[cid * 2, pl.ds(row0, ROWS_PER_TILE)])
    pltpu.sync_copy(zeros_hbm.at[pl.ds(row0, ROWS_PER_TILE)],
                    acc_sh.at[pl.ds(row0, ROWS_PER_TILE)])
    plsc.subcore_barrier()

    pltpu.sync_copy(ones_hbm, rows0_v)
    dd = (dst0_v, dst1_v, dst2_v, dst3_v)
    sd = (si0, si1, sg0, sg1)
    ss = (ss0, ss1, ss2, ss3)

    def fire_d(g, b):
        off = ebase + g * CHUNK
        pltpu.async_copy(edst_hbm.at[pl.ds(off, CHUNK)], dd[b], sd[b])

    def wait_d(b):
        pltpu.make_async_copy(edst_hbm.at[pl.ds(0, CHUNK)], dd[b],
                              sd[b]).wait()

    def wait_scat(b):
        pltpu.make_async_copy(rows0_v, acc_sh.at[dd[b]], ss[b]).wait()

    fire_d(0, 0)
    fire_d(1, 1)

    def step2(g, b):
        bp2 = (b + 2) % 4

        @pl.when(g >= 2)
        def _():
            wait_scat(bp2)

        @pl.when(g + 2 < NUM_CHUNKS)
        def _():
            fire_d(g + 2, bp2)

        wait_d(b)
        pltpu.async_copy(rows0_v, acc_sh.at[dd[b]], ss[b], add=True)

    def outer2(o, carry):
        for k in range(4):
            step2(4 * o + k, k)
        return carry

    main2 = 4 * (NUM_CHUNKS // 4)
    lax.fori_loop(0, NUM_CHUNKS // 4, outer2, 0)
    for g in range(main2, NUM_CHUNKS):
        b = g % 4
        bp2 = (b + 2) % 4
        if g >= 2:
            wait_scat(bp2)
        if g + 2 < NUM_CHUNKS:
            fire_d(g + 2, bp2)
        wait_d(b)
        pltpu.async_copy(rows0_v, acc_sh.at[dd[b]], ss[b], add=True)
    wait_scat((NUM_CHUNKS - 2) % 4)
    wait_scat((NUM_CHUNKS - 1) % 4)

    plsc.subcore_barrier()
    pltpu.sync_copy(acc_sh.at[pl.ds(row0, ROWS_PER_TILE)],
                    out_hbm.at[cid * 2 + 1, pl.ds(row0, ROWS_PER_TILE)])


def _sc_aggregate(table, edge_src, edge_dst, zeros, ones):
    mesh = plsc.VectorSubcoreMesh(core_axis_name="c", subcore_axis_name="s")
    return pl.kernel(
        _agg_body,
        out_type=jax.ShapeDtypeStruct((2 * NUM_CORES, N_ACC, HID),
                                      jnp.float32),
        mesh=mesh,
        scratch_types=(
            [pltpu.VMEM((CHUNK,), jnp.int32)] * 6
            + [pltpu.VMEM((CHUNK, HID), jnp.float32)] * 2
            + [pltpu.VMEM_SHARED((N_ACC, HID), jnp.float32)]
            + [pltpu.SemaphoreType.DMA] * 8
        ),
    )(table, edge_src, edge_dst, zeros, ones)


def _upd_body(nf_ref, f0_ref, f1_ref, d0_ref, d1_ref, w1a_ref, w1b_ref,
              b1_ref, w2_ref, b2_ref, out_ref):
    feat = f0_ref[0] + f1_ref[0]
    deg = d0_ref[0][:, 0:1] + d1_ref[0][:, 0:1]
    agg = feat / jnp.maximum(deg, 1.0)
    nf = nf_ref[...]
    h = (jnp.dot(nf, w1a_ref[...], preferred_element_type=jnp.float32)
         + jnp.dot(agg, w1b_ref[...], preferred_element_type=jnp.float32)
         + b1_ref[...])
    h = jnp.maximum(h, 0.0)
    h = jnp.dot(h, w2_ref[...], preferred_element_type=jnp.float32) + b2_ref[...]
    h = jnp.maximum(h, 0.0)
    out_ref[...] = nf + h


def _update(nf, acc4, w1, b1, w2, b2, block=1000):
    n = nf.shape[0]
    return pl.pallas_call(
        _upd_body,
        grid=(n // block,),
        in_specs=[
            pl.BlockSpec((block, HID), lambda i: (i, 0)),
            pl.BlockSpec((1, block, HID), lambda i: (0, i, 0)),
            pl.BlockSpec((1, block, HID), lambda i: (2, i, 0)),
            pl.BlockSpec((1, block, HID), lambda i: (1, i, 0)),
            pl.BlockSpec((1, block, HID), lambda i: (3, i, 0)),
            pl.BlockSpec((HID, HID), lambda i: (0, 0)),
            pl.BlockSpec((HID, HID), lambda i: (0, 0)),
            pl.BlockSpec((HID,), lambda i: (0,)),
            pl.BlockSpec((HID, HID), lambda i: (0, 0)),
            pl.BlockSpec((HID,), lambda i: (0,)),
        ],
        out_specs=pl.BlockSpec((block, HID), lambda i: (i, 0)),
        out_shape=jax.ShapeDtypeStruct((n, HID), jnp.float32),
    )(nf, acc4, acc4, acc4, acc4, w1[:HID], w1[HID:], b1, w2, b2)


def kernel(lane_points, agent_history, edge_lane_lane, edge_agent_agent,
           edge_lane_agent, lW1, lB1, lW2, lB2, lW3, lB3,
           aW1, aB1, aW2, aB2, aW3, aB3,
           llW1, llB1, llW2, llB2,
           aaW1, aaB1, aaW2, aaB2,
           laW1, laB1, laW2, laB2):
    zeros = jnp.zeros((N_ACC, HID), jnp.float32)
    ones = jnp.ones((CHUNK, HID), jnp.float32)

    def pad_edges(e):
        src = jnp.concatenate([e[0], jnp.zeros((E_PAD,), jnp.int32)])
        junk = N_NODE + jnp.arange(E_PAD, dtype=jnp.int32) % (N_ACC - N_NODE)
        dst = jnp.concatenate([e[1], junk])
        return src, dst

    lane_feat = _encode(lane_points, lW1, lB1, lW2, lB2, lW3, lB3, block=400)
    agent_feat = _encode(agent_history, aW1, aB1, aW2, aB2, aW3, aB3,
                         block=400)

    acc_ll = _sc_aggregate(lane_feat, *pad_edges(edge_lane_lane), zeros, ones)
    lane_feat2 = _update(lane_feat, acc_ll, llW1, llB1, llW2, llB2)

    acc_aa = _sc_aggregate(agent_feat, *pad_edges(edge_agent_agent),
                           zeros, ones)
    agent_feat2 = _update(agent_feat, acc_aa, aaW1, aaB1, aaW2, aaB2)

    acc_la = _sc_aggregate(lane_feat2, *pad_edges(edge_lane_agent),
                           zeros, ones)
    agent_feat3 = _update(agent_feat2, acc_la, laW1, laB1, laW2, laB2)

    return lane_feat2, agent_feat3


# speedup vs baseline: 1.4139x; 1.4139x over previous
import functools

import jax
import jax.numpy as jnp
from jax import lax
from jax.experimental import pallas as pl
from jax.experimental.pallas import tpu as pltpu
from jax.experimental.pallas import tpu_sc as plsc

N_NODE = 10000
HID = 128
E_TOTAL = 320000
NUM_CORES = 2
NUM_SUBCORES = 16
NUM_WORKERS = NUM_CORES * NUM_SUBCORES
ROWS_PER_TILE = 632
N_ACC = NUM_SUBCORES * ROWS_PER_TILE
CHUNK = 80
EDGES_PER_WORKER = E_TOTAL // NUM_WORKERS
NUM_CHUNKS = EDGES_PER_WORKER // CHUNK


def _enc_body(x_ref, w1_ref, b1_ref, w2_ref, b2_ref, w3_ref, b3_ref, out_ref,
              *, n_pts):
    x = x_ref[...]
    c = x[:, n_pts - 1:n_pts, :]
    col = lax.broadcasted_iota(jnp.int32, (1, 1, x.shape[-1]), 2)
    x = x - jnp.where(col < 2, c, 0.0)
    w1 = w1_ref[...]
    b1 = b1_ref[...]
    w2 = w2_ref[...]
    b2 = b2_ref[...]
    m = None
    for p in range(n_pts):
        h = jnp.dot(x[:, p, :], w1, preferred_element_type=jnp.float32) + b1
        h = jnp.maximum(h, 0.0)
        h = jnp.dot(h, w2, preferred_element_type=jnp.float32) + b2
        h = jnp.maximum(h, 0.0)
        m = h if m is None else jnp.maximum(m, h)
    z = jnp.dot(m, w3_ref[...], preferred_element_type=jnp.float32) + b3_ref[...]
    out_ref[...] = jnp.maximum(z, 0.0)


def _encode(x, w1, b1, w2, b2, w3, b3, block):
    n, n_pts, f_in = x.shape
    return pl.pallas_call(
        functools.partial(_enc_body, n_pts=n_pts),
        grid=(n // block,),
        in_specs=[
            pl.BlockSpec((block, n_pts, f_in), lambda i: (i, 0, 0)),
            pl.BlockSpec((f_in, HID), lambda i: (0, 0)),
            pl.BlockSpec((HID,), lambda i: (0,)),
            pl.BlockSpec((HID, HID), lambda i: (0, 0)),
            pl.BlockSpec((HID,), lambda i: (0,)),
            pl.BlockSpec((HID, HID), lambda i: (0, 0)),
            pl.BlockSpec((HID,), lambda i: (0,)),
        ],
        out_specs=pl.BlockSpec((block, HID), lambda i: (i, 0)),
        out_shape=jax.ShapeDtypeStruct((n, HID), jnp.float32),
    )(x, w1, b1, w2, b2, w3, b3)


def _agg_body(table_hbm, esrc_hbm, edst_hbm, zeros_hbm, ones_hbm, out_hbm,
              src0_v, src1_v, dst0_v, dst1_v, dst2_v, dst3_v, rows0_v,
              rows1_v, acc_sh, si0, si1, sg0, sg1, ss0, ss1, ss2, ss3):
    cid = lax.axis_index("c")
    sid = lax.axis_index("s")
    wid = sid * NUM_CORES + cid
    row0 = sid * ROWS_PER_TILE
    ebase = wid * EDGES_PER_WORKER

    src_v = (src0_v, src1_v)
    dst_v = (dst0_v, dst1_v)
    rows_v = (rows0_v, rows1_v)
    si = (si0, si1)
    sg = (sg0, sg1)

    def fire_idx(g, b, with_src):
        off = ebase + g * CHUNK
        if with_src:
            pltpu.async_copy(esrc_hbm.at[pl.ds(off, CHUNK)], src_v[b], si[b])
        pltpu.async_copy(edst_hbm.at[pl.ds(off, CHUNK)], dst_v[b], si[b])

    def wait_idx(b, with_src):
        if with_src:
            pltpu.make_async_copy(esrc_hbm.at[pl.ds(0, CHUNK)], src_v[b],
                                  si[b]).wait()
        pltpu.make_async_copy(edst_hbm.at[pl.ds(0, CHUNK)], dst_v[b],
                              si[b]).wait()

    def fire_gather(b):
        pltpu.async_copy(table_hbm.at[src_v[b]], rows_v[b], sg[b])

    def wait_gather(b):
        pltpu.make_async_copy(table_hbm.at[src_v[b]], rows_v[b], sg[b]).wait()

    pltpu.sync_copy(zeros_hbm.at[pl.ds(row0, ROWS_PER_TILE)],
                    acc_sh.at[pl.ds(row0, ROWS_PER_TILE)])
    plsc.subcore_barrier()

    fire_idx(0, 0, True)
    wait_idx(0, True)
    fire_gather(0)
    fire_idx(1, 1, True)

    def step1(g, b):
        nb = 1 - b
        wait_idx(nb, True)
        fire_gather(nb)
        wait_gather(b)
        pltpu.sync_copy(rows_v[b], acc_sh.at[dst_v[b]], add=True)

        @pl.when(g + 2 < NUM_CHUNKS)
        def _():
            fire_idx(g + 2, b, True)

    def outer1(o, carry):
        step1(2 * o, 0)
        step1(2 * o + 1, 1)
        return carry

    lax.fori_loop(0, (NUM_CHUNKS - 1) // 2, outer1, 0)
    last = (NUM_CHUNKS - 1) % 2
    wait_gather(last)
    pltpu.sync_copy(rows_v[last], acc_sh.at[dst_v[last]], add=True)

    plsc.subcore_barrier()
    pltpu.sync_copy(acc_sh.at[pl.ds(row0, ROWS_PER_TILE)],
                    out_hbm.at[cid * 2, pl.ds(row0, ROWS_PER_TILE)])
    pltpu.sync_copy(zeros_hbm.at[pl.ds(row0, ROWS_PER_TILE)],
                    acc_sh.at[pl.ds(row0, ROWS_PER_TILE)])
    plsc.subcore_barrier()

    pltpu.sync_copy(ones_hbm, rows0_v)
    dd = (dst0_v, dst1_v, dst2_v, dst3_v)
    sd = (si0, si1, sg0, sg1)
    ss = (ss0, ss1, ss2, ss3)

    def fire_d(g, b):
        off = ebase + g * CHUNK
        pltpu.async_copy(edst_hbm.at[pl.ds(off, CHUNK)], dd[b], sd[b])

    def wait_d(b):
        pltpu.make_async_copy(edst_hbm.at[pl.ds(0, CHUNK)], dd[b],
                              sd[b]).wait()

    def wait_scat(b):
        pltpu.make_async_copy(rows0_v, acc_sh.at[dd[b]], ss[b]).wait()

    fire_d(0, 0)
    fire_d(1, 1)

    def step2(g, b):
        bp2 = (b + 2) % 4

        @pl.when(g >= 2)
        def _():
            wait_scat(bp2)

        @pl.when(g + 2 < NUM_CHUNKS)
        def _():
            fire_d(g + 2, bp2)

        wait_d(b)
        pltpu.async_copy(rows0_v, acc_sh.at[dd[b]], ss[b], add=True)

    def outer2(o, carry):
        for k in range(4):
            step2(4 * o + k, k)
        return carry

    main2 = 4 * (NUM_CHUNKS // 4)
    lax.fori_loop(0, NUM_CHUNKS // 4, outer2, 0)
    for g in range(main2, NUM_CHUNKS):
        b = g % 4
        bp2 = (b + 2) % 4
        if g >= 2:
            wait_scat(bp2)
        if g + 2 < NUM_CHUNKS:
            fire_d(g + 2, bp2)
        wait_d(b)
        pltpu.async_copy(rows0_v, acc_sh.at[dd[b]], ss[b], add=True)
    wait_scat((NUM_CHUNKS - 2) % 4)
    wait_scat((NUM_CHUNKS - 1) % 4)

    plsc.subcore_barrier()
    pltpu.sync_copy(acc_sh.at[pl.ds(row0, ROWS_PER_TILE)],
                    out_hbm.at[cid * 2 + 1, pl.ds(row0, ROWS_PER_TILE)])


def _sc_aggregate(table, edge_src, edge_dst, zeros, ones):
    mesh = plsc.VectorSubcoreMesh(core_axis_name="c", subcore_axis_name="s")
    return pl.kernel(
        _agg_body,
        out_type=jax.ShapeDtypeStruct((2 * NUM_CORES, N_ACC, HID),
                                      jnp.float32),
        mesh=mesh,
        scratch_types=(
            [pltpu.VMEM((CHUNK,), jnp.int32)] * 6
            + [pltpu.VMEM((CHUNK, HID), jnp.float32)] * 2
            + [pltpu.VMEM_SHARED((N_ACC, HID), jnp.float32)]
            + [pltpu.SemaphoreType.DMA] * 8
        ),
    )(table, edge_src, edge_dst, zeros, ones)


def _upd_body(nf_ref, f0_ref, f1_ref, d0_ref, d1_ref, w1a_ref, w1b_ref,
              b1_ref, w2_ref, b2_ref, out_ref):
    feat = f0_ref[0] + f1_ref[0]
    deg = d0_ref[0][:, 0:1] + d1_ref[0][:, 0:1]
    agg = feat / jnp.maximum(deg, 1.0)
    nf = nf_ref[...]
    h = (jnp.dot(nf, w1a_ref[...], preferred_element_type=jnp.float32)
         + jnp.dot(agg, w1b_ref[...], preferred_element_type=jnp.float32)
         + b1_ref[...])
    h = jnp.maximum(h, 0.0)
    h = jnp.dot(h, w2_ref[...], preferred_element_type=jnp.float32) + b2_ref[...]
    h = jnp.maximum(h, 0.0)
    out_ref[...] = nf + h


def _update(nf, acc4, w1, b1, w2, b2, block=1000):
    n = nf.shape[0]
    return pl.pallas_call(
        _upd_body,
        grid=(n // block,),
        in_specs=[
            pl.BlockSpec((block, HID), lambda i: (i, 0)),
            pl.BlockSpec((1, block, HID), lambda i: (0, i, 0)),
            pl.BlockSpec((1, block, HID), lambda i: (2, i, 0)),
            pl.BlockSpec((1, block, HID), lambda i: (1, i, 0)),
            pl.BlockSpec((1, block, HID), lambda i: (3, i, 0)),
            pl.BlockSpec((HID, HID), lambda i: (0, 0)),
            pl.BlockSpec((HID, HID), lambda i: (0, 0)),
            pl.BlockSpec((HID,), lambda i: (0,)),
            pl.BlockSpec((HID, HID), lambda i: (0, 0)),
            pl.BlockSpec((HID,), lambda i: (0,)),
        ],
        out_specs=pl.BlockSpec((block, HID), lambda i: (i, 0)),
        out_shape=jax.ShapeDtypeStruct((n, HID), jnp.float32),
    )(nf, acc4, acc4, acc4, acc4, w1[:HID], w1[HID:], b1, w2, b2)


def kernel(lane_points, agent_history, edge_lane_lane, edge_agent_agent,
           edge_lane_agent, lW1, lB1, lW2, lB2, lW3, lB3,
           aW1, aB1, aW2, aB2, aW3, aB3,
           llW1, llB1, llW2, llB2,
           aaW1, aaB1, aaW2, aaB2,
           laW1, laB1, laW2, laB2):
    zeros = jnp.zeros((N_ACC, HID), jnp.float32)
    ones = jnp.ones((CHUNK, HID), jnp.float32)

    def pad_edges(e):
        return e[0], e[1]

    lane_feat = _encode(lane_points, lW1, lB1, lW2, lB2, lW3, lB3, block=400)
    agent_feat = _encode(agent_history, aW1, aB1, aW2, aB2, aW3, aB3,
                         block=400)

    acc_ll = _sc_aggregate(lane_feat, *pad_edges(edge_lane_lane), zeros, ones)
    lane_feat2 = _update(lane_feat, acc_ll, llW1, llB1, llW2, llB2)

    acc_aa = _sc_aggregate(agent_feat, *pad_edges(edge_agent_agent),
                           zeros, ones)
    agent_feat2 = _update(agent_feat, acc_aa, aaW1, aaB1, aaW2, aaB2)

    acc_la = _sc_aggregate(lane_feat2, *pad_edges(edge_lane_agent),
                           zeros, ones)
    agent_feat3 = _update(agent_feat2, acc_la, laW1, laB1, laW2, laB2)

    return lane_feat2, agent_feat3
